# Initial kernel scaffold; baseline (speedup 1.0000x reference)
import functools

import jax
import jax.numpy as jnp
from jax import lax
from jax.experimental import pallas as pl
from jax.experimental.pallas import tpu as pltpu
from jax.experimental.pallas import tpu_sc as plsc

N = 10000
E = 320000
D = 128

NC = 2
NS = 16
NW = NC * NS

CHUNK = 128
CPT = 79
EPW = CPT * CHUNK
EPAD = NW * EPW
TOT_CHUNKS = NW * CPT

NP = 10240
RPT = NP // NS
LANES = 16

_sc_mesh = plsc.VectorSubcoreMesh(core_axis_name="c", subcore_axis_name="s")


@functools.partial(
    pl.kernel,
    out_type=jax.ShapeDtypeStruct((NC * NP,), jnp.float32),
    mesh=_sc_mesh,
    scratch_types=[
        pltpu.VMEM((CPT, CHUNK), jnp.int32),
        pltpu.VMEM((NP,), jnp.float32),
        pltpu.VMEM((RPT,), jnp.float32),
        pltpu.VMEM((RPT,), jnp.float32),
        pltpu.VMEM_SHARED((NS, NP), jnp.float32),
    ],
)
def _sc_deg(dst_hbm, out_hbm, idxb, hist, tbuf, accv, sall):
    cid = lax.axis_index("c")
    sid = lax.axis_index("s")
    wid = cid * NS + sid
    zeros16 = jnp.zeros((LANES,), jnp.float32)
    ones16 = jnp.ones((LANES,), jnp.float32)

    pltpu.sync_copy(dst_hbm.at[pl.ds(wid * CPT, CPT)], idxb)

    def _zero(i, _):
        hist[pl.ds(i * LANES, LANES)] = zeros16
        return 0
    lax.fori_loop(0, NP // LANES, _zero, 0)

    def _hist_chunk(c, _):
        def _hist_vec(j, _):
            idxv = idxb[c, pl.ds(j * LANES, LANES)]
            plsc.addupdate_scatter(hist, [idxv], ones16)
            return 0
        lax.fori_loop(0, CHUNK // LANES, _hist_vec, 0)
        return 0
    lax.fori_loop(0, CPT, _hist_chunk, 0)

    pltpu.sync_copy(hist, sall.at[sid])
    plsc.subcore_barrier()

    def _zacc(i, _):
        accv[pl.ds(i * LANES, LANES)] = zeros16
        return 0
    lax.fori_loop(0, RPT // LANES, _zacc, 0)

    def _red(t, _):
        pltpu.sync_copy(sall.at[t, pl.ds(sid * RPT, RPT)], tbuf)
        def _add(i, _):
            s = pl.ds(i * LANES, LANES)
            accv[s] = accv[s] + tbuf[s]
            return 0
        lax.fori_loop(0, RPT // LANES, _add, 0)
        return 0
    lax.fori_loop(0, NS, _red, 0)

    pltpu.sync_copy(accv, out_hbm.at[pl.ds(cid * NP + sid * RPT, RPT)])


@functools.partial(
    pl.kernel,
    out_type=jax.ShapeDtypeStruct((NC * NP, D), jnp.float32),
    mesh=_sc_mesh,
    scratch_types=[
        pltpu.VMEM((CPT, CHUNK), jnp.int32),
        pltpu.VMEM((CPT, CHUNK), jnp.int32),
        pltpu.VMEM((CHUNK, D), jnp.float32),
        pltpu.VMEM_SHARED((NP, D), jnp.float32),
        pltpu.SemaphoreType.DMA,
    ],
)
def _sc_gs(hp_hbm, src_hbm, dst_hbm, z_hbm, out_hbm, srcb, dstb, rows, acc, sem):
    cid = lax.axis_index("c")
    sid = lax.axis_index("s")
    wid = cid * NS + sid
    rs = pl.ds(sid * RPT, RPT)

    pltpu.sync_copy(z_hbm, acc.at[rs])
    pltpu.sync_copy(src_hbm.at[pl.ds(wid * CPT, CPT)], srcb)
    pltpu.sync_copy(dst_hbm.at[pl.ds(wid * CPT, CPT)], dstb)
    plsc.subcore_barrier()

    def _chunk(c, _):
        pltpu.async_copy(hp_hbm.at[srcb.at[c]], rows, sem).wait()
        pltpu.sync_copy(rows, acc.at[dstb.at[c]], add=True)
        return 0
    lax.fori_loop(0, CPT, _chunk, 0)

    plsc.subcore_barrier()
    pltpu.sync_copy(acc.at[rs], out_hbm.at[pl.ds(cid * NP + sid * RPT, RPT)])


def _dis(degT_ref):
    return lax.rsqrt(degT_ref[:, 0:1] + degT_ref[:, 1:2] + 1.0)


def _tc_first_body(x_ref, w_ref, degT_ref, out_ref):
    dis = _dis(degT_ref)
    t = jnp.dot(x_ref[...], w_ref[...], preferred_element_type=jnp.float32)
    out_ref[0:N, :] = t * dis[0:N]
    out_ref[N:NP, :] = jnp.zeros((NP - N, D), jnp.float32)


_tc_first = pl.pallas_call(
    _tc_first_body, out_shape=jax.ShapeDtypeStruct((NP, D), jnp.float32))


def _tc_mid_body(hp_ref, p_ref, degT_ref, b_ref, w_ref, out_ref):
    dis = _dis(degT_ref)
    s = p_ref[0:N, :] + p_ref[NP:NP + N, :] + hp_ref[0:N, :]
    z = jnp.maximum(s * dis[0:N] + b_ref[...], 0.0)
    t = jnp.dot(z, w_ref[...], preferred_element_type=jnp.float32)
    out_ref[0:N, :] = t * dis[0:N]
    out_ref[N:NP, :] = jnp.zeros((NP - N, D), jnp.float32)


_tc_mid = pl.pallas_call(
    _tc_mid_body, out_shape=jax.ShapeDtypeStruct((NP, D), jnp.float32))


def _tc_last_body(hp_ref, p_ref, degT_ref, b_ref, out_ref):
    dis = _dis(degT_ref)
    s = p_ref[0:N, :] + p_ref[NP:NP + N, :] + hp_ref[0:N, :]
    out_ref[...] = s * dis[0:N] + b_ref[...]


_tc_last = pl.pallas_call(
    _tc_last_body, out_shape=jax.ShapeDtypeStruct((N, D), jnp.float32))


def kernel(x, edge_index, W1, b1, W2, b2, W3, b3):
    src = edge_index[0].astype(jnp.int32)
    dst = edge_index[1].astype(jnp.int32)
    pad = jnp.full((EPAD - E,), N, dtype=jnp.int32)
    src2 = jnp.concatenate([src, pad]).reshape(TOT_CHUNKS, CHUNK)
    dst2 = jnp.concatenate([dst, pad]).reshape(TOT_CHUNKS, CHUNK)
    zrows = jnp.zeros((RPT, D), jnp.float32)

    degp = _sc_deg(dst2)
    degT = jnp.stack([degp[:NP], degp[NP:]], axis=1)
    b1r = b1.reshape(1, D)
    b2r = b2.reshape(1, D)
    b3r = b3.reshape(1, D)

    hp1 = _tc_first(x, W1, degT)
    p1 = _sc_gs(hp1, src2, dst2, zrows)
    hp2 = _tc_mid(hp1, p1, degT, b1r, W2)
    p2 = _sc_gs(hp2, src2, dst2, zrows)
    hp3 = _tc_mid(hp2, p2, degT, b2r, W3)
    return _tc_last(hp3, _sc_gs(hp3, src2, dst2, zrows), degT, b3r)


# speedup vs baseline: 8.3441x; 8.3441x over previous
import functools

import jax
import jax.numpy as jnp
from jax import lax
from jax.experimental import pallas as pl
from jax.experimental.pallas import tpu as pltpu
from jax.experimental.pallas import tpu_sc as plsc

N = 10000
E = 320000
D = 128

NC = 2
NS = 16
NW = NC * NS

CHUNK = 128
CPT = 80
EPW = CPT * CHUNK
EPAD = NW * EPW
TOT_CHUNKS = NW * CPT

NP = 10240
RPT = NP // NS
LANES = 16

_sc_mesh = plsc.VectorSubcoreMesh(core_axis_name="c", subcore_axis_name="s")


DW = 16


@functools.partial(
    pl.kernel,
    out_type=jax.ShapeDtypeStruct((NC * NP, DW), jnp.float32),
    mesh=_sc_mesh,
    scratch_types=[
        pltpu.VMEM((CPT, CHUNK), jnp.int32),
        pltpu.VMEM((CHUNK, DW), jnp.float32),
        pltpu.VMEM_SHARED((NP, DW), jnp.float32),
    ],
)
def _sc_deg(dst_hbm, z_hbm, out_hbm, idxb, onesb, sdeg):
    cid = lax.axis_index("c")
    sid = lax.axis_index("s")
    wid = cid * NS + sid
    rs = pl.ds(sid * RPT, RPT)
    ones16 = jnp.ones((LANES,), jnp.float32)

    pltpu.sync_copy(dst_hbm.at[pl.ds(wid * CPT, CPT)], idxb)

    def _fill(i, _):
        onesb[i, :] = ones16
        return 0
    lax.fori_loop(0, CHUNK, _fill, 0)

    pltpu.sync_copy(z_hbm, sdeg.at[rs])
    plsc.subcore_barrier()

    def _chunk(c, _):
        pltpu.sync_copy(onesb, sdeg.at[idxb.at[c]], add=True)
        return 0
    lax.fori_loop(0, CPT, _chunk, 0)

    plsc.subcore_barrier()
    pltpu.sync_copy(sdeg.at[rs], out_hbm.at[pl.ds(cid * NP + sid * RPT, RPT)])


@functools.partial(
    pl.kernel,
    out_type=jax.ShapeDtypeStruct((NC * NP, D), jnp.float32),
    mesh=_sc_mesh,
    scratch_types=[
        pltpu.VMEM((CPT, CHUNK), jnp.int32),
        pltpu.VMEM((CPT, CHUNK), jnp.int32),
        pltpu.VMEM((CHUNK, D), jnp.float32),
        pltpu.VMEM_SHARED((NP, D), jnp.float32),
        pltpu.SemaphoreType.DMA,
    ],
)
def _sc_gs(hp_hbm, src_hbm, dst_hbm, z_hbm, out_hbm, srcb, dstb, rows, acc, sem):
    cid = lax.axis_index("c")
    sid = lax.axis_index("s")
    wid = cid * NS + sid
    rs = pl.ds(sid * RPT, RPT)

    pltpu.sync_copy(z_hbm, acc.at[rs])
    pltpu.sync_copy(src_hbm.at[pl.ds(wid * CPT, CPT)], srcb)
    pltpu.sync_copy(dst_hbm.at[pl.ds(wid * CPT, CPT)], dstb)
    plsc.subcore_barrier()

    def _chunk(c, _):
        pltpu.async_copy(hp_hbm.at[srcb.at[c]], rows, sem).wait()
        pltpu.sync_copy(rows, acc.at[dstb.at[c]], add=True)
        return 0
    lax.fori_loop(0, CPT, _chunk, 0)

    plsc.subcore_barrier()
    pltpu.sync_copy(acc.at[rs], out_hbm.at[pl.ds(cid * NP + sid * RPT, RPT)])


def _dis(degT_ref):
    return lax.rsqrt(degT_ref[:, 0:1] + degT_ref[:, 1:2] + 1.0)


def _tc_first_body(x_ref, w_ref, degT_ref, out_ref):
    dis = _dis(degT_ref)
    t = jnp.dot(x_ref[...], w_ref[...], preferred_element_type=jnp.float32)
    out_ref[0:N, :] = t * dis[0:N]
    out_ref[N:NP, :] = jnp.zeros((NP - N, D), jnp.float32)


_tc_first = pl.pallas_call(
    _tc_first_body, out_shape=jax.ShapeDtypeStruct((NP, D), jnp.float32))


def _tc_mid_body(hp_ref, p_ref, degT_ref, b_ref, w_ref, out_ref):
    dis = _dis(degT_ref)
    s = p_ref[0:N, :] + p_ref[NP:NP + N, :] + hp_ref[0:N, :]
    z = jnp.maximum(s * dis[0:N] + b_ref[...], 0.0)
    t = jnp.dot(z, w_ref[...], preferred_element_type=jnp.float32)
    out_ref[0:N, :] = t * dis[0:N]
    out_ref[N:NP, :] = jnp.zeros((NP - N, D), jnp.float32)


_tc_mid = pl.pallas_call(
    _tc_mid_body, out_shape=jax.ShapeDtypeStruct((NP, D), jnp.float32))


def _tc_last_body(hp_ref, p_ref, degT_ref, b_ref, out_ref):
    dis = _dis(degT_ref)
    s = p_ref[0:N, :] + p_ref[NP:NP + N, :] + hp_ref[0:N, :]
    out_ref[...] = s * dis[0:N] + b_ref[...]


_tc_last = pl.pallas_call(
    _tc_last_body, out_shape=jax.ShapeDtypeStruct((N, D), jnp.float32))


def kernel(x, edge_index, W1, b1, W2, b2, W3, b3):
    src = edge_index[0].astype(jnp.int32)
    dst = edge_index[1].astype(jnp.int32)
    pad = jnp.full((EPAD - E,), N, dtype=jnp.int32)
    src2 = jnp.concatenate([src, pad]).reshape(TOT_CHUNKS, CHUNK)
    dst2 = jnp.concatenate([dst, pad]).reshape(TOT_CHUNKS, CHUNK)
    zrows = jnp.zeros((RPT, D), jnp.float32)
    zdeg = jnp.zeros((RPT, DW), jnp.float32)

    degp = _sc_deg(dst2, zdeg)[:, 0]
    degT = jnp.stack([degp[:NP], degp[NP:]], axis=1)
    b1r = b1.reshape(1, D)
    b2r = b2.reshape(1, D)
    b3r = b3.reshape(1, D)

    hp1 = _tc_first(x, W1, degT)
    p1 = _sc_gs(hp1, src2, dst2, zrows)
    hp2 = _tc_mid(hp1, p1, degT, b1r, W2)
    p2 = _sc_gs(hp2, src2, dst2, zrows)
    hp3 = _tc_mid(hp2, p2, degT, b2r, W3)
    return _tc_last(hp3, _sc_gs(hp3, src2, dst2, zrows), degT, b3r)


# speedup vs baseline: 28.1639x; 3.3753x over previous
import functools

import jax
import jax.numpy as jnp
from jax import lax
from jax.experimental import pallas as pl
from jax.experimental.pallas import tpu as pltpu
from jax.experimental.pallas import tpu_sc as plsc

N = 10000
E = 320000
D = 128

NC = 2
NS = 16
NW = NC * NS

CHUNK = 128
CPT = 80
EPW = CPT * CHUNK
EPAD = NW * EPW
TOT_CHUNKS = NW * CPT

NP = 10240
RPT = NP // NS
LANES = 16

_sc_mesh = plsc.VectorSubcoreMesh(core_axis_name="c", subcore_axis_name="s")


DW = 16


@functools.partial(
    pl.kernel,
    out_type=jax.ShapeDtypeStruct((NC * NP, DW), jnp.float32),
    mesh=_sc_mesh,
    scratch_types=[
        pltpu.VMEM((CPT, CHUNK), jnp.int32),
        pltpu.VMEM((CHUNK, DW), jnp.float32),
        pltpu.VMEM_SHARED((NP, DW), jnp.float32),
    ],
)
def _sc_deg(dst_hbm, z_hbm, out_hbm, idxb, onesb, sdeg):
    cid = lax.axis_index("c")
    sid = lax.axis_index("s")
    wid = cid * NS + sid
    rs = pl.ds(sid * RPT, RPT)
    ones16 = jnp.ones((LANES,), jnp.float32)

    pltpu.sync_copy(dst_hbm.at[pl.ds(wid * CPT, CPT)], idxb)

    def _fill(i, _):
        onesb[i, :] = ones16
        return 0
    lax.fori_loop(0, CHUNK, _fill, 0)

    pltpu.sync_copy(z_hbm, sdeg.at[rs])
    plsc.subcore_barrier()

    def _chunk(c, _):
        pltpu.sync_copy(onesb, sdeg.at[idxb.at[c]], add=True)
        return 0
    lax.fori_loop(0, CPT, _chunk, 0)

    plsc.subcore_barrier()
    pltpu.sync_copy(sdeg.at[rs], out_hbm.at[pl.ds(cid * NP + sid * RPT, RPT)])


@functools.partial(
    pl.kernel,
    out_type=jax.ShapeDtypeStruct((NC * NP, D), jnp.float32),
    mesh=_sc_mesh,
    scratch_types=[
        pltpu.VMEM((CPT, CHUNK), jnp.int32),
        pltpu.VMEM((CPT // 2, CHUNK), jnp.int32),
        pltpu.VMEM((CHUNK, D), jnp.float32),
        pltpu.VMEM((CHUNK, D), jnp.float32),
        pltpu.VMEM_SHARED((NP, D), jnp.float32),
        pltpu.SemaphoreType.DMA,
        pltpu.SemaphoreType.DMA,
    ],
)
def _sc_gs(hp_hbm, src_hbm, dst_hbm, z_hbm, out_hbm, srcb, dstb,
           rows0, rows1, acc, sem0, sem1):
    cid = lax.axis_index("c")
    sid = lax.axis_index("s")
    wid = cid * NS + sid
    rs = pl.ds(sid * RPT, RPT)
    half = CPT // 2

    pltpu.sync_copy(z_hbm, acc.at[rs])
    pltpu.sync_copy(src_hbm.at[pl.ds(wid * CPT, CPT)], srcb)
    pltpu.sync_copy(dst_hbm.at[pl.ds(wid * CPT, half)], dstb)
    plsc.subcore_barrier()

    pltpu.async_copy(hp_hbm.at[srcb.at[0]], rows0, sem0)
    pltpu.async_copy(hp_hbm.at[srcb.at[1]], rows1, sem1)

    def _pair(dst_base):
        def body(i, _):
            c = 2 * i
            pltpu.make_async_copy(hp_hbm.at[srcb.at[c]], rows0, sem0).wait()
            pltpu.sync_copy(rows0, acc.at[dstb.at[c - dst_base]], add=True)
            pltpu.async_copy(hp_hbm.at[srcb.at[c + 2]], rows0, sem0)
            pltpu.make_async_copy(hp_hbm.at[srcb.at[c + 1]], rows1, sem1).wait()
            pltpu.sync_copy(rows1, acc.at[dstb.at[c + 1 - dst_base]], add=True)
            pltpu.async_copy(hp_hbm.at[srcb.at[c + 3]], rows1, sem1)
            return 0
        return body

    lax.fori_loop(0, half // 2, _pair(0), 0)
    pltpu.sync_copy(dst_hbm.at[pl.ds(wid * CPT + half, half)], dstb)
    lax.fori_loop(half // 2, CPT // 2 - 1, _pair(half), 0)

    pltpu.make_async_copy(hp_hbm.at[srcb.at[CPT - 2]], rows0, sem0).wait()
    pltpu.sync_copy(rows0, acc.at[dstb.at[half - 2]], add=True)
    pltpu.make_async_copy(hp_hbm.at[srcb.at[CPT - 1]], rows1, sem1).wait()
    pltpu.sync_copy(rows1, acc.at[dstb.at[half - 1]], add=True)

    plsc.subcore_barrier()
    pltpu.sync_copy(acc.at[rs], out_hbm.at[pl.ds(cid * NP + sid * RPT, RPT)])


def _dis(degT_ref):
    return lax.rsqrt(degT_ref[:, 0:1] + degT_ref[:, 1:2] + 1.0)


def _tc_first_body(x_ref, w_ref, degT_ref, out_ref):
    dis = _dis(degT_ref)
    t = jnp.dot(x_ref[...], w_ref[...], preferred_element_type=jnp.float32)
    out_ref[0:N, :] = t * dis[0:N]
    out_ref[N:NP, :] = jnp.zeros((NP - N, D), jnp.float32)


_tc_first = pl.pallas_call(
    _tc_first_body, out_shape=jax.ShapeDtypeStruct((NP, D), jnp.float32))


def _tc_mid_body(hp_ref, p_ref, degT_ref, b_ref, w_ref, out_ref):
    dis = _dis(degT_ref)
    s = p_ref[0:N, :] + p_ref[NP:NP + N, :] + hp_ref[0:N, :]
    z = jnp.maximum(s * dis[0:N] + b_ref[...], 0.0)
    t = jnp.dot(z, w_ref[...], preferred_element_type=jnp.float32)
    out_ref[0:N, :] = t * dis[0:N]
    out_ref[N:NP, :] = jnp.zeros((NP - N, D), jnp.float32)


_tc_mid = pl.pallas_call(
    _tc_mid_body, out_shape=jax.ShapeDtypeStruct((NP, D), jnp.float32))


def _tc_last_body(hp_ref, p_ref, degT_ref, b_ref, out_ref):
    dis = _dis(degT_ref)
    s = p_ref[0:N, :] + p_ref[NP:NP + N, :] + hp_ref[0:N, :]
    out_ref[...] = s * dis[0:N] + b_ref[...]


_tc_last = pl.pallas_call(
    _tc_last_body, out_shape=jax.ShapeDtypeStruct((N, D), jnp.float32))


def kernel(x, edge_index, W1, b1, W2, b2, W3, b3):
    src = edge_index[0].astype(jnp.int32)
    dst = edge_index[1].astype(jnp.int32)
    pad = N + jnp.arange(EPAD - E, dtype=jnp.int32) % (NP - N)
    src2 = jnp.concatenate([src, pad]).reshape(TOT_CHUNKS, CHUNK)
    dst2 = jnp.concatenate([dst, pad]).reshape(TOT_CHUNKS, CHUNK)
    zrows = jnp.zeros((RPT, D), jnp.float32)
    zdeg = jnp.zeros((RPT, DW), jnp.float32)

    degp = _sc_deg(dst2, zdeg)[:, 0]
    degT = jnp.stack([degp[:NP], degp[NP:]], axis=1)
    b1r = b1.reshape(1, D)
    b2r = b2.reshape(1, D)
    b3r = b3.reshape(1, D)

    hp1 = _tc_first(x, W1, degT)
    p1 = _sc_gs(hp1, src2, dst2, zrows)
    hp2 = _tc_mid(hp1, p1, degT, b1r, W2)
    p2 = _sc_gs(hp2, src2, dst2, zrows)
    hp3 = _tc_mid(hp2, p2, degT, b2r, W3)
    return _tc_last(hp3, _sc_gs(hp3, src2, dst2, zrows), degT, b3r)
